# baseline (device time: 15072 ns/iter reference)
import jax
import jax.numpy as jnp
from jax import lax
from jax.experimental import pallas as pl
from jax.experimental.pallas import tpu as pltpu

N_DEV = 4
N_HALF = 2
D_ORDER = (2, 1, 3)


def kernel(A, B):
    m, k = A.shape
    _, n = B.shape
    m_out = m // N_DEV
    n_h = n // N_HALF

    def body(a_ref, b_ref, out_ref, av_ref, bv_ref, stage_ref, comm_ref,
             send_sems, recv_sems, local_sems):
        my_pos = lax.axis_index("i")

        barrier_sem = pltpu.get_barrier_semaphore()
        for d in range(1, N_DEV):
            peer = lax.rem(my_pos + d, N_DEV)
            pl.semaphore_signal(
                barrier_sem, inc=1,
                device_id=(peer,), device_id_type=pl.DeviceIdType.MESH,
            )

        cp_a = pltpu.make_async_copy(a_ref, av_ref, local_sems.at[0])
        cp_b0 = pltpu.make_async_copy(
            b_ref.at[:, pl.ds(0, n_h)], bv_ref.at[:, pl.ds(0, n_h)],
            local_sems.at[1],
        )
        cp_b1 = pltpu.make_async_copy(
            b_ref.at[:, pl.ds(n_h, n_h)], bv_ref.at[:, pl.ds(n_h, n_h)],
            local_sems.at[2],
        )
        cp_a.start()
        cp_b0.start()
        cp_b1.start()
        cp_a.wait()
        cp_b0.wait()

        rdmas = []
        first = True
        for d in D_ORDER:
            tgt = lax.rem(my_pos + d, N_DEV)
            for h in range(N_HALF):
                if d == D_ORDER[0] and h == 1:
                    cp_b1.wait()
                stage_ref[d - 1, h, :, :] = jnp.dot(
                    av_ref[pl.ds(tgt * m_out, m_out), :],
                    bv_ref[:, pl.ds(h * n_h, n_h)],
                    preferred_element_type=jnp.float32,
                ).astype(jnp.bfloat16)
                if first:
                    pl.semaphore_wait(barrier_sem, N_DEV - 1)
                    first = False
                s = (d - 1) * N_HALF + h
                rdma = pltpu.make_async_remote_copy(
                    src_ref=stage_ref.at[d - 1, h],
                    dst_ref=comm_ref.at[d - 1, h],
                    send_sem=send_sems.at[s],
                    recv_sem=recv_sems.at[s],
                    device_id=(tgt,),
                    device_id_type=pl.DeviceIdType.MESH,
                )
                rdma.start()
                rdmas.append(((d, h), rdma))

        for h in range(N_HALF):
            out_ref[:, pl.ds(h * n_h, n_h)] = jnp.dot(
                av_ref[pl.ds(my_pos * m_out, m_out), :],
                bv_ref[:, pl.ds(h * n_h, n_h)],
                preferred_element_type=jnp.float32,
            )

        for (d, h), rdma in rdmas:
            rdma.wait()
            sl = pl.ds(h * n_h, n_h)
            out_ref[:, sl] = (
                out_ref[:, sl] + comm_ref[d - 1, h, :, :].astype(jnp.float32)
            )

    return pl.pallas_call(
        body,
        out_shape=jax.ShapeDtypeStruct((m_out, n), jnp.float32),
        in_specs=[
            pl.BlockSpec(memory_space=pl.ANY),
            pl.BlockSpec(memory_space=pl.ANY),
        ],
        out_specs=pl.BlockSpec(memory_space=pltpu.VMEM),
        scratch_shapes=[
            pltpu.VMEM((m, k), jnp.float32),
            pltpu.VMEM((k, n), jnp.float32),
            pltpu.VMEM((N_DEV - 1, N_HALF, m_out, n_h), jnp.bfloat16),
            pltpu.VMEM((N_DEV - 1, N_HALF, m_out, n_h), jnp.bfloat16),
            pltpu.SemaphoreType.DMA(((N_DEV - 1) * N_HALF,)),
            pltpu.SemaphoreType.DMA(((N_DEV - 1) * N_HALF,)),
            pltpu.SemaphoreType.DMA((3,)),
        ],
        compiler_params=pltpu.CompilerParams(collective_id=0),
    )(A, B)


# device time: 14812 ns/iter; 1.0176x vs baseline; 1.0176x over previous
import jax
import jax.numpy as jnp
from jax import lax
from jax.experimental import pallas as pl
from jax.experimental.pallas import tpu as pltpu

N_DEV = 4
N_HALF = 2
D_ORDER = (2, 1, 3)


def kernel(A, B):
    m, k = A.shape
    _, n = B.shape
    m_out = m // N_DEV
    n_h = n // N_HALF

    def body(a_ref, b_ref, out_ref, stage_ref, comm_ref,
             send_sems, recv_sems):
        my_pos = lax.axis_index("i")

        barrier_sem = pltpu.get_barrier_semaphore()
        for d in range(1, N_DEV):
            peer = lax.rem(my_pos + d, N_DEV)
            pl.semaphore_signal(
                barrier_sem, inc=1,
                device_id=(peer,), device_id_type=pl.DeviceIdType.MESH,
            )

        rdmas = []
        for d in D_ORDER:
            tgt = lax.rem(my_pos + d, N_DEV)
            for h in range(N_HALF):
                stage_ref[d - 1, h, :, :] = jnp.dot(
                    a_ref[pl.ds(tgt * m_out, m_out), :],
                    b_ref[:, pl.ds(h * n_h, n_h)],
                    preferred_element_type=jnp.float32,
                ).astype(jnp.bfloat16)
                if d == D_ORDER[0] and h == 0:
                    pl.semaphore_wait(barrier_sem, N_DEV - 1)
                s = (d - 1) * N_HALF + h
                rdma = pltpu.make_async_remote_copy(
                    src_ref=stage_ref.at[d - 1, h],
                    dst_ref=comm_ref.at[d - 1, h],
                    send_sem=send_sems.at[s],
                    recv_sem=recv_sems.at[s],
                    device_id=(tgt,),
                    device_id_type=pl.DeviceIdType.MESH,
                )
                rdma.start()
                rdmas.append(((d, h), rdma))

        for h in range(N_HALF):
            out_ref[:, pl.ds(h * n_h, n_h)] = jnp.dot(
                a_ref[pl.ds(my_pos * m_out, m_out), :],
                b_ref[:, pl.ds(h * n_h, n_h)],
                preferred_element_type=jnp.float32,
            )

        for (d, h), rdma in rdmas:
            rdma.wait()
            sl = pl.ds(h * n_h, n_h)
            out_ref[:, sl] = (
                out_ref[:, sl] + comm_ref[d - 1, h, :, :].astype(jnp.float32)
            )

    return pl.pallas_call(
        body,
        out_shape=jax.ShapeDtypeStruct((m_out, n), jnp.float32),
        in_specs=[
            pl.BlockSpec(memory_space=pltpu.VMEM),
            pl.BlockSpec(memory_space=pltpu.VMEM),
        ],
        out_specs=pl.BlockSpec(memory_space=pltpu.VMEM),
        scratch_shapes=[
            pltpu.VMEM((N_DEV - 1, N_HALF, m_out, n_h), jnp.bfloat16),
            pltpu.VMEM((N_DEV - 1, N_HALF, m_out, n_h), jnp.bfloat16),
            pltpu.SemaphoreType.DMA(((N_DEV - 1) * N_HALF,)),
            pltpu.SemaphoreType.DMA(((N_DEV - 1) * N_HALF,)),
        ],
        compiler_params=pltpu.CompilerParams(collective_id=0),
    )(A, B)
